# baseline (speedup 1.0000x reference)
import functools

import jax
import jax.numpy as jnp
import numpy as np
from jax import lax
from jax.experimental import pallas as pl
from jax.experimental.pallas import tpu as pltpu
from jax.experimental.pallas import tpu_sc as plsc

N_POINTS = 65536
N_LEVELS = 16
N_FEATS = 4
TABLE_ROWS = 1 << 19
MASK19 = np.int32(TABLE_ROWS - 1)
P1 = np.int32(np.int64(2654435761) - (1 << 32))
P2 = np.int32(805459861)

NC, NS, L = 2, 16, 16
NW = NC * NS
CHUNK = N_POINTS // NW
TILE = 128
NTILES = CHUNK // TILE
PV = TILE // L
T0_GROUPS = 1024

_mesh = plsc.VectorSubcoreMesh(core_axis_name="c", subcore_axis_name="s")


@functools.partial(
    pl.kernel,
    out_type=jax.ShapeDtypeStruct((N_POINTS, N_LEVELS * N_FEATS), jnp.float32),
    mesh=_mesh,
    compiler_params=pltpu.CompilerParams(needs_layout_passes=False,
                                         use_tc_tiling_on_sc=False,
                                         disable_bounds_checks=True),
    scratch_types=[
        pltpu.VMEM((CHUNK, 3), jnp.float32),
        pltpu.VMEM((3, CHUNK), jnp.float32),
        pltpu.VMEM((T0_GROUPS, 16), jnp.float32),
        pltpu.VMEM((2, 8, TILE), jnp.int32),
        pltpu.VMEM((2, 8, TILE), jnp.int32),
        pltpu.VMEM((2, 8, TILE), jnp.float32),
        pltpu.VMEM((2, 8, TILE, 16), jnp.float32),
        pltpu.VMEM((TILE, N_LEVELS * N_FEATS), jnp.float32),
        pltpu.SemaphoreType.DMA,
        pltpu.SemaphoreType.DMA,
        pltpu.SemaphoreType.DMA,
    ],
)
def _hash_embed(coords_hbm, table_hbm, out_hbm, coords_raw, coords_v, tab0_v,
                idx_v, sub_v, wv, rows_v, out_v, semA, semB, semO):
    wid = lax.axis_index("s") * NC + lax.axis_index("c")
    base = wid * CHUNK
    pltpu.async_copy(coords_hbm.at[pl.ds(base, CHUNK), :], coords_raw, semO)
    pltpu.sync_copy(table_hbm.at[pl.ds(0, T0_GROUPS)], tab0_v)
    pltpu.make_async_copy(coords_hbm.at[pl.ds(base, CHUNK), :], coords_raw,
                          semO).wait()

    iv = lax.iota(jnp.int32, L)

    def _trans_body(j, _):
        p = iv + j * L
        for d in range(3):
            coords_v[d, pl.ds(j * L, L)] = plsc.load_gather(
                coords_raw, [p, jnp.broadcast_to(jnp.int32(d), (L,))])
        return 0

    lax.fori_loop(0, CHUNK // L, _trans_body, 0)
    sems = (semA, semB)

    def splat_i(s):
        return jnp.broadcast_to(jnp.int32(s), (L,))

    def splat_f(s):
        return jnp.broadcast_to(jnp.float32(s), (L,))

    def grid_coords(v, scale_v):
        pos = v * scale_v + 0.5
        vi = pos.astype(jnp.int32)
        fr = pos - vi.astype(jnp.float32)
        return vi, fr

    def load_xyz(tb, jL):
        x = coords_v[0, pl.ds(tb + jL, L)]
        y = coords_v[1, pl.ds(tb + jL, L)]
        z = coords_v[2, pl.ds(tb + jL, L)]
        return x, y, z

    def level0_local(tb):
        def jbody(j, _):
            jL = j * L
            p_vec = iv + jL
            x, y, z = load_xyz(tb, jL)
            sc = splat_f(15.0)
            xi, fx = grid_coords(x, sc)
            yi, fy = grid_coords(y, sc)
            zi, fz = grid_coords(z, sc)
            sx = (splat_f(1.0) - fx, fx)
            sy = (splat_f(1.0) - fy, fy)
            sz = (splat_f(1.0) - fz, fz)
            ax = (xi, xi + 1)
            by0 = lax.shift_left(yi, 4)
            cz0 = lax.shift_left(zi, 8)
            by = (by0, by0 + 16)
            cz = (cz0, cz0 + 256)
            accs = [splat_f(0.0) for _ in range(N_FEATS)]
            for dx in (0, 1):
                for dy in (0, 1):
                    for dz in (0, 1):
                        h = (ax[dx] + by[dy] + cz[dz]) & splat_i(4095)
                        g = lax.shift_right_logical(h, 2)
                        s = lax.shift_left(h & 3, 2)
                        w = (sx[dx] * sy[dy]) * sz[dz]
                        for f in range(N_FEATS):
                            val = plsc.load_gather(tab0_v, [g, s + f])
                            accs[f] = accs[f] + w * val
            for f in range(N_FEATS):
                plsc.store_scatter(out_v, [p_vec, splat_i(f)], accs[f])
            return 0

        lax.fori_loop(0, PV, jbody, 0)

    def build_fire(tb, l, b, degen):
        res = jnp.int32(16) << l
        scale_v = jnp.broadcast_to(res.astype(jnp.float32) - 1.0, (L,))
        resv = jnp.broadcast_to(res, (L,))
        if not degen:
            lin = jnp.broadcast_to(l < 3, (L,))
            maskv = jnp.where(lin, resv * resv * resv - 1, splat_i(MASK19))
            myv = jnp.where(lin, resv, splat_i(P1))
            mzv = jnp.where(lin, resv * resv, splat_i(P2))

        def jbody(j, _):
            jL = j * L
            x, y, z = load_xyz(tb, jL)
            xi, fx = grid_coords(x, scale_v)
            yi, fy = grid_coords(y, scale_v)
            sx = (splat_f(1.0) - fx, fx)
            sy = (splat_f(1.0) - fy, fy)
            if degen:
                ax = (xi, xi + 1)
                by0 = yi * resv
                by = (by0, by0 + resv)
                for dx in (0, 1):
                    for dy in (0, 1):
                        c = dx * 2 + dy
                        h = (ax[dx] + by[dy]) & splat_i(MASK19)
                        idx_v[b, c, pl.ds(jL, L)] = \
                            lax.shift_right_logical(h, 2)
                        sub_v[b, c, pl.ds(jL, L)] = lax.shift_left(h & 3, 2)
                        wv[b, c, pl.ds(jL, L)] = sx[dx] * sy[dy]
            else:
                zi, fz = grid_coords(z, scale_v)
                sz = (splat_f(1.0) - fz, fz)
                ax = (xi, xi + 1)
                by0, cz0 = yi * myv, zi * mzv
                by = (by0, by0 + myv)
                cz = (cz0, cz0 + mzv)
                for dx in (0, 1):
                    for dy in (0, 1):
                        for dz in (0, 1):
                            c = dx * 4 + dy * 2 + dz
                            hl = ax[dx] + by[dy] + cz[dz]
                            hx = ax[dx] ^ by[dy] ^ cz[dz]
                            h = jnp.where(lin, hl, hx) & maskv
                            idx_v[b, c, pl.ds(jL, L)] = \
                                lax.shift_right_logical(h, 2)
                            sub_v[b, c, pl.ds(jL, L)] = \
                                lax.shift_left(h & 3, 2)
                            wv[b, c, pl.ds(jL, L)] = (sx[dx] * sy[dy]) * sz[dz]
            return 0

        lax.fori_loop(0, PV, jbody, 0)
        for c in range(4 if degen else 8):
            pltpu.async_copy(table_hbm.at[idx_v.at[b, c]], rows_v.at[b, c],
                             sems[b])

    def build_fire15(tb, b):
        scale_v = splat_f(float((16 << 15) - 1))

        def jbody(j, _):
            jL = j * L
            x = coords_v[0, pl.ds(tb + jL, L)]
            xi, fx = grid_coords(x, scale_v)
            for dx in (0, 1):
                h = (xi + dx) & splat_i(MASK19)
                idx_v[b, dx, pl.ds(jL, L)] = lax.shift_right_logical(h, 2)
                sub_v[b, dx, pl.ds(jL, L)] = lax.shift_left(h & 3, 2)
            wv[b, 0, pl.ds(jL, L)] = splat_f(1.0) - fx
            wv[b, 1, pl.ds(jL, L)] = fx
            return 0

        lax.fori_loop(0, PV, jbody, 0)
        for c in range(2):
            pltpu.async_copy(table_hbm.at[idx_v.at[b, c]], rows_v.at[b, c],
                             sems[b])

    def combine(lvl, b, ncorners):
        for c in range(ncorners):
            pltpu.make_async_copy(table_hbm.at[idx_v.at[b, c]],
                                  rows_v.at[b, c], sems[b]).wait()
        col0 = lvl * N_FEATS
        for j in range(PV):
            jL = j * L
            p_vec = iv + jL
            accs = [splat_f(0.0) for _ in range(N_FEATS)]
            for c in range(ncorners):
                w = wv[b, c, pl.ds(jL, L)]
                sub = sub_v[b, c, pl.ds(jL, L)]
                rc = rows_v.at[b, c]
                for f in range(N_FEATS):
                    val = plsc.load_gather(rc, [p_vec, sub + f])
                    accs[f] = accs[f] + w * val
            for f in range(N_FEATS):
                col = jnp.broadcast_to(col0 + f, (L,))
                plsc.store_scatter(out_v, [p_vec, col], accs[f])

    def tile_body(t, _):
        tb = t * TILE

        @pl.when(t > 0)
        def _():
            pltpu.make_async_copy(
                out_v, out_hbm.at[pl.ds(base + (t - 1) * TILE, TILE), :],
                semO).wait()

        build_fire(tb, jnp.int32(1), 0, False)
        level0_local(tb)

        def pair_body(k, _):
            l = 2 * k + 1
            build_fire(tb, l + 1, 1, False)
            combine(l, 0, 8)
            build_fire(tb, l + 2, 0, False)
            combine(l + 1, 1, 8)
            return 0

        lax.fori_loop(0, 5, pair_body, 0)

        build_fire(tb, jnp.int32(12), 1, True)
        combine(jnp.int32(11), 0, 8)

        build_fire(tb, jnp.int32(13), 0, True)
        combine(jnp.int32(12), 1, 4)
        build_fire(tb, jnp.int32(14), 1, True)
        combine(jnp.int32(13), 0, 4)
        build_fire15(tb, 0)
        combine(jnp.int32(14), 1, 4)
        combine(jnp.int32(15), 0, 2)
        pltpu.async_copy(out_v, out_hbm.at[pl.ds(base + tb, TILE), :], semO)
        return 0

    lax.fori_loop(0, NTILES, tile_body, 0)
    pltpu.make_async_copy(
        out_v, out_hbm.at[pl.ds(base + (NTILES - 1) * TILE, TILE), :],
        semO).wait()


def kernel(coords, params):
    table = params[: TABLE_ROWS * N_FEATS].reshape(TABLE_ROWS // 4, 16)
    return _hash_embed(coords.astype(jnp.float32), table)


# speedup vs baseline: 1.1726x; 1.1726x over previous
import functools

import jax
import jax.numpy as jnp
import numpy as np
from jax import lax
from jax.experimental import pallas as pl
from jax.experimental.pallas import tpu as pltpu
from jax.experimental.pallas import tpu_sc as plsc

N_POINTS = 65536
N_LEVELS = 16
N_FEATS = 4
TABLE_ROWS = 1 << 19
MASK19 = np.int32(TABLE_ROWS - 1)
P1 = np.int32(np.int64(2654435761) - (1 << 32))
P2 = np.int32(805459861)

NC, NS, L = 2, 16, 16
NW = NC * NS
CHUNK = N_POINTS // NW
TILE = 128
NTILES = CHUNK // TILE
PV = TILE // L
T0_GROUPS = 1024

_mesh = plsc.VectorSubcoreMesh(core_axis_name="c", subcore_axis_name="s")


@functools.partial(
    pl.kernel,
    out_type=jax.ShapeDtypeStruct((N_POINTS, N_LEVELS * N_FEATS), jnp.float32),
    mesh=_mesh,
    compiler_params=pltpu.CompilerParams(needs_layout_passes=False,
                                         use_tc_tiling_on_sc=False,
                                         disable_bounds_checks=True),
    scratch_types=[
        pltpu.VMEM((3, CHUNK), jnp.float32),
        pltpu.VMEM((T0_GROUPS, 16), jnp.float32),
        pltpu.VMEM((3, 8, TILE), jnp.int32),
        pltpu.VMEM((3, 8, TILE), jnp.int32),
        pltpu.VMEM((3, 8, TILE), jnp.float32),
        pltpu.VMEM((3, 8, TILE, 16), jnp.float32),
        pltpu.VMEM((TILE, N_LEVELS * N_FEATS), jnp.float32),
        pltpu.SemaphoreType.DMA,
        pltpu.SemaphoreType.DMA,
        pltpu.SemaphoreType.DMA,
        pltpu.SemaphoreType.DMA,
    ],
)
def _hash_embed(coords_hbm, table_hbm, out_hbm, coords_v, tab0_v,
                idx_v, sub_v, wv, rows_v, out_v, semA, semB, semC, semO):
    wid = lax.axis_index("s") * NC + lax.axis_index("c")
    base = wid * CHUNK
    pltpu.async_copy(coords_hbm.at[:, pl.ds(base, CHUNK)], coords_v, semO)
    pltpu.sync_copy(table_hbm.at[pl.ds(0, T0_GROUPS)], tab0_v)
    pltpu.make_async_copy(coords_hbm.at[:, pl.ds(base, CHUNK)], coords_v,
                          semO).wait()

    iv = lax.iota(jnp.int32, L)
    sems = (semA, semB, semC)

    def splat_i(s):
        return jnp.broadcast_to(jnp.int32(s), (L,))

    def splat_f(s):
        return jnp.broadcast_to(jnp.float32(s), (L,))

    def grid_coords(v, scale_v):
        pos = v * scale_v + 0.5
        vi = pos.astype(jnp.int32)
        fr = pos - vi.astype(jnp.float32)
        return vi, fr

    def load_xyz(tb, jL):
        x = coords_v[0, pl.ds(tb + jL, L)]
        y = coords_v[1, pl.ds(tb + jL, L)]
        z = coords_v[2, pl.ds(tb + jL, L)]
        return x, y, z

    def level0_local(tb):
        def jbody(j, _):
            jL = j * L
            p_vec = iv + jL
            x, y, z = load_xyz(tb, jL)
            sc = splat_f(15.0)
            xi, fx = grid_coords(x, sc)
            yi, fy = grid_coords(y, sc)
            zi, fz = grid_coords(z, sc)
            sx = (splat_f(1.0) - fx, fx)
            sy = (splat_f(1.0) - fy, fy)
            sz = (splat_f(1.0) - fz, fz)
            ax = (xi, xi + 1)
            by0 = lax.shift_left(yi, 4)
            cz0 = lax.shift_left(zi, 8)
            by = (by0, by0 + 16)
            cz = (cz0, cz0 + 256)
            accs = [splat_f(0.0) for _ in range(N_FEATS)]
            for dx in (0, 1):
                for dy in (0, 1):
                    for dz in (0, 1):
                        h = (ax[dx] + by[dy] + cz[dz]) & splat_i(4095)
                        g = lax.shift_right_logical(h, 2)
                        s = lax.shift_left(h & 3, 2)
                        w = (sx[dx] * sy[dy]) * sz[dz]
                        for f in range(N_FEATS):
                            val = plsc.load_gather(tab0_v, [g, s + f])
                            accs[f] = accs[f] + w * val
            for f in range(N_FEATS):
                plsc.store_scatter(out_v, [p_vec, splat_i(f)], accs[f])
            return 0

        lax.fori_loop(0, PV, jbody, 0)

    def build_fire(tb, l, b, degen):
        res = jnp.int32(16) << l
        scale_v = jnp.broadcast_to(res.astype(jnp.float32) - 1.0, (L,))
        resv = jnp.broadcast_to(res, (L,))
        if not degen:
            lin = jnp.broadcast_to(l < 3, (L,))
            maskv = jnp.where(lin, resv * resv * resv - 1, splat_i(MASK19))
            myv = jnp.where(lin, resv, splat_i(P1))
            mzv = jnp.where(lin, resv * resv, splat_i(P2))

        def jbody(j, _):
            jL = j * L
            x, y, z = load_xyz(tb, jL)
            xi, fx = grid_coords(x, scale_v)
            yi, fy = grid_coords(y, scale_v)
            sx = (splat_f(1.0) - fx, fx)
            sy = (splat_f(1.0) - fy, fy)
            if degen:
                ax = (xi, xi + 1)
                by0 = yi * resv
                by = (by0, by0 + resv)
                for dx in (0, 1):
                    for dy in (0, 1):
                        c = dx * 2 + dy
                        h = (ax[dx] + by[dy]) & splat_i(MASK19)
                        idx_v[b, c, pl.ds(jL, L)] = \
                            lax.shift_right_logical(h, 2)
                        sub_v[b, c, pl.ds(jL, L)] = lax.shift_left(h & 3, 2)
                        wv[b, c, pl.ds(jL, L)] = sx[dx] * sy[dy]
            else:
                zi, fz = grid_coords(z, scale_v)
                sz = (splat_f(1.0) - fz, fz)
                ax = (xi, xi + 1)
                by0, cz0 = yi * myv, zi * mzv
                by = (by0, by0 + myv)
                cz = (cz0, cz0 + mzv)
                for dx in (0, 1):
                    for dy in (0, 1):
                        for dz in (0, 1):
                            c = dx * 4 + dy * 2 + dz
                            hl = ax[dx] + by[dy] + cz[dz]
                            hx = ax[dx] ^ by[dy] ^ cz[dz]
                            h = jnp.where(lin, hl, hx) & maskv
                            idx_v[b, c, pl.ds(jL, L)] = \
                                lax.shift_right_logical(h, 2)
                            sub_v[b, c, pl.ds(jL, L)] = \
                                lax.shift_left(h & 3, 2)
                            wv[b, c, pl.ds(jL, L)] = (sx[dx] * sy[dy]) * sz[dz]
            return 0

        lax.fori_loop(0, PV, jbody, 0)
        for c in range(4 if degen else 8):
            pltpu.async_copy(table_hbm.at[idx_v.at[b, c]], rows_v.at[b, c],
                             sems[b])

    def build_fire15(tb, b):
        scale_v = splat_f(float((16 << 15) - 1))

        def jbody(j, _):
            jL = j * L
            x = coords_v[0, pl.ds(tb + jL, L)]
            xi, fx = grid_coords(x, scale_v)
            for dx in (0, 1):
                h = (xi + dx) & splat_i(MASK19)
                idx_v[b, dx, pl.ds(jL, L)] = lax.shift_right_logical(h, 2)
                sub_v[b, dx, pl.ds(jL, L)] = lax.shift_left(h & 3, 2)
            wv[b, 0, pl.ds(jL, L)] = splat_f(1.0) - fx
            wv[b, 1, pl.ds(jL, L)] = fx
            return 0

        lax.fori_loop(0, PV, jbody, 0)
        for c in range(2):
            pltpu.async_copy(table_hbm.at[idx_v.at[b, c]], rows_v.at[b, c],
                             sems[b])

    def combine(lvl, b, ncorners):
        for c in range(ncorners):
            pltpu.make_async_copy(table_hbm.at[idx_v.at[b, c]],
                                  rows_v.at[b, c], sems[b]).wait()
        col0 = lvl * N_FEATS
        for j in range(PV):
            jL = j * L
            p_vec = iv + jL
            accs = [splat_f(0.0) for _ in range(N_FEATS)]
            for c in range(ncorners):
                w = wv[b, c, pl.ds(jL, L)]
                sub = sub_v[b, c, pl.ds(jL, L)]
                rc = rows_v.at[b, c]
                for f in range(N_FEATS):
                    val = plsc.load_gather(rc, [p_vec, sub + f])
                    accs[f] = accs[f] + w * val
            for f in range(N_FEATS):
                col = jnp.broadcast_to(col0 + f, (L,))
                plsc.store_scatter(out_v, [p_vec, col], accs[f])

    def tile_body(t, _):
        tb = t * TILE

        @pl.when(t > 0)
        def _():
            pltpu.make_async_copy(
                out_v, out_hbm.at[pl.ds(base + (t - 1) * TILE, TILE), :],
                semO).wait()

        build_fire(tb, jnp.int32(1), 0, False)
        build_fire(tb, jnp.int32(2), 1, False)
        level0_local(tb)

        def tri_body(k, _):
            l = 3 * k + 1
            build_fire(tb, l + 2, 2, False)
            combine(l, 0, 8)
            build_fire(tb, l + 3, 0, False)
            combine(l + 1, 1, 8)
            build_fire(tb, l + 4, 1, False)
            combine(l + 2, 2, 8)
            return 0

        lax.fori_loop(0, 3, tri_body, 0)

        build_fire(tb, jnp.int32(12), 2, True)
        combine(jnp.int32(10), 0, 8)
        build_fire(tb, jnp.int32(13), 0, True)
        combine(jnp.int32(11), 1, 8)
        build_fire(tb, jnp.int32(14), 1, True)
        combine(jnp.int32(12), 2, 4)
        build_fire15(tb, 2)
        combine(jnp.int32(13), 0, 4)
        combine(jnp.int32(14), 1, 4)
        combine(jnp.int32(15), 2, 2)
        pltpu.async_copy(out_v, out_hbm.at[pl.ds(base + tb, TILE), :], semO)
        return 0

    lax.fori_loop(0, NTILES, tile_body, 0)
    pltpu.make_async_copy(
        out_v, out_hbm.at[pl.ds(base + (NTILES - 1) * TILE, TILE), :],
        semO).wait()


def kernel(coords, params):
    coords_t = coords.astype(jnp.float32).T
    table = params[: TABLE_ROWS * N_FEATS].reshape(TABLE_ROWS // 4, 16)
    return _hash_embed(coords_t, table)
